# baseline (speedup 1.0000x reference)
import jax
import jax.numpy as jnp
from jax import lax
from jax.experimental import pallas as pl

_BLK = 4096
_OH = 1024


def _take_rows(table, idx):
    return lax.gather(
        table, idx[:, None],
        dimension_numbers=lax.GatherDimensionNumbers(
            offset_dims=(1,), collapsed_slice_dims=(0,), start_index_map=(0,)),
        slice_sizes=(1, table.shape[1]),
        mode=lax.GatherScatterMode.PROMISE_IN_BOUNDS)


def _mlp_body(wv, uv, pid, lid, nf, dt,
              w1w, w1u, pos_t, lang_t, w1p, w1l, w1n, b1, w2, b2,
              p_out, h_out):
    f32 = jnp.float32
    acc = jnp.dot(wv[...], w1w[...], preferred_element_type=f32)
    acc += jnp.dot(uv[...], w1u[...], preferred_element_type=f32)
    acc += jnp.dot(nf[...], w1n[...], preferred_element_type=f32)
    bf16 = jnp.bfloat16
    iota = lax.broadcasted_iota(jnp.int32, (_BLK, _OH), 1)
    poh = (iota == pid[...]).astype(bf16)
    loh = (iota == lid[...]).astype(bf16)
    pv = jnp.dot(poh, pos_t[...].astype(bf16), preferred_element_type=f32)
    lv = jnp.dot(loh, lang_t[...].astype(bf16), preferred_element_type=f32)
    acc += jnp.dot(pv, w1p[...], preferred_element_type=f32)
    acc += jnp.dot(lv, w1l[...], preferred_element_type=f32)
    h1 = jnp.maximum(acc + b1[...], 0.0)
    dp = jnp.sum(h1 * w2[...], axis=1, keepdims=True) + b2[...]
    dp = jnp.clip(dp, -6.58, 8.1)
    h = jnp.clip(jnp.exp2(dp), 0.0104, 274.0)
    p = jnp.clip(jnp.exp2(-dt[...] / h), 0.0001, 0.9999)
    p_out[...] = p
    h_out[...] = h


def kernel(word_id, user_idx, pos_id, lang_id, num_features, delta_t,
           word_table, user_table, pos_table, lang_table, W1, b1, W2, b2):
    B = word_id.shape[0]
    f32 = jnp.float32

    wv = _take_rows(word_table, word_id)
    uv = _take_rows(user_table, user_idx)

    pid2 = pos_id.reshape(B, 1)
    lid2 = lang_id.reshape(B, 1)
    nf8 = jnp.pad(num_features, ((0, 0), (0, 3)))
    dt2 = delta_t.reshape(B, 1)
    pos_t = jnp.pad(pos_table, ((0, _OH - pos_table.shape[0]), (0, 0)))
    lang_t = jnp.pad(lang_table, ((0, _OH - lang_table.shape[0]), (0, 0)))
    w1w = W1[0:32]
    w1u = W1[32:64]
    w1p = W1[64:72]
    w1l = W1[72:80]
    w1n = jnp.pad(W1[80:85], ((0, 3), (0, 0)))
    b1r = b1.reshape(1, 64)
    w2r = W2.reshape(1, 64)
    b2r = b2.reshape(1, 1)

    row = lambda d: pl.BlockSpec((_BLK, d), lambda i: (i, 0))
    full = lambda s: pl.BlockSpec(s, lambda i: (0, 0))
    p2, h2 = pl.pallas_call(
        _mlp_body,
        grid=(B // _BLK,),
        in_specs=[
            row(32), row(32), row(1), row(1), row(8), row(1),
            full((32, 64)), full((32, 64)), full((_OH, 8)), full((_OH, 8)),
            full((8, 64)), full((8, 64)), full((8, 64)), full((1, 64)),
            full((1, 64)), full((1, 1)),
        ],
        out_specs=[row(1), row(1)],
        out_shape=[
            jax.ShapeDtypeStruct((B, 1), f32),
            jax.ShapeDtypeStruct((B, 1), f32),
        ],
    )(wv, uv, pid2, lid2, nf8, dt2,
      w1w, w1u, pos_t, lang_t, w1p, w1l, w1n, b1r, w2r, b2r)

    return p2.reshape(B), h2.reshape(B)


# speedup vs baseline: 1.0187x; 1.0187x over previous
import jax
import jax.numpy as jnp
from jax import lax
from jax.experimental import pallas as pl

_BLK = 2048
_OH = 1024


def _take_rows(table, idx):
    return lax.gather(
        table, idx[:, None],
        dimension_numbers=lax.GatherDimensionNumbers(
            offset_dims=(1,), collapsed_slice_dims=(0,), start_index_map=(0,)),
        slice_sizes=(1, table.shape[1]),
        mode=lax.GatherScatterMode.PROMISE_IN_BOUNDS)


def _mlp_body(wv, uv, pid, lid, nf, dt,
              w1w, w1u, pos_t, lang_t, w1p, w1l, w1n, b1, w2, b2,
              p_out, h_out):
    f32 = jnp.float32
    acc = jnp.dot(wv[...], w1w[...], preferred_element_type=f32)
    acc += jnp.dot(uv[...], w1u[...], preferred_element_type=f32)
    acc += jnp.dot(nf[...], w1n[...], preferred_element_type=f32)
    bf16 = jnp.bfloat16
    iota = lax.broadcasted_iota(jnp.int32, (_BLK, _OH), 1)
    poh = (iota == pid[...]).astype(bf16)
    loh = (iota == lid[...]).astype(bf16)
    pv = jnp.dot(poh, pos_t[...].astype(bf16), preferred_element_type=f32)
    lv = jnp.dot(loh, lang_t[...].astype(bf16), preferred_element_type=f32)
    acc += jnp.dot(pv, w1p[...], preferred_element_type=f32)
    acc += jnp.dot(lv, w1l[...], preferred_element_type=f32)
    h1 = jnp.maximum(acc + b1[...], 0.0)
    dp = jnp.sum(h1 * w2[...], axis=1, keepdims=True) + b2[...]
    dp = jnp.clip(dp, -6.58, 8.1)
    h = jnp.clip(jnp.exp2(dp), 0.0104, 274.0)
    p = jnp.clip(jnp.exp2(-dt[...] / h), 0.0001, 0.9999)
    p_out[...] = p
    h_out[...] = h


def kernel(word_id, user_idx, pos_id, lang_id, num_features, delta_t,
           word_table, user_table, pos_table, lang_table, W1, b1, W2, b2):
    B = word_id.shape[0]
    f32 = jnp.float32

    wv = _take_rows(word_table, word_id)
    uv = _take_rows(user_table, user_idx)

    pid2 = pos_id.reshape(B, 1)
    lid2 = lang_id.reshape(B, 1)
    nf8 = jnp.pad(num_features, ((0, 0), (0, 3)))
    dt2 = delta_t.reshape(B, 1)
    pos_t = jnp.pad(pos_table, ((0, _OH - pos_table.shape[0]), (0, 0)))
    lang_t = jnp.pad(lang_table, ((0, _OH - lang_table.shape[0]), (0, 0)))
    w1w = W1[0:32]
    w1u = W1[32:64]
    w1p = W1[64:72]
    w1l = W1[72:80]
    w1n = jnp.pad(W1[80:85], ((0, 3), (0, 0)))
    b1r = b1.reshape(1, 64)
    w2r = W2.reshape(1, 64)
    b2r = b2.reshape(1, 1)

    row = lambda d: pl.BlockSpec((_BLK, d), lambda i: (i, 0))
    full = lambda s: pl.BlockSpec(s, lambda i: (0, 0))
    p2, h2 = pl.pallas_call(
        _mlp_body,
        grid=(B // _BLK,),
        in_specs=[
            row(32), row(32), row(1), row(1), row(8), row(1),
            full((32, 64)), full((32, 64)), full((_OH, 8)), full((_OH, 8)),
            full((8, 64)), full((8, 64)), full((8, 64)), full((1, 64)),
            full((1, 64)), full((1, 1)),
        ],
        out_specs=[row(1), row(1)],
        out_shape=[
            jax.ShapeDtypeStruct((B, 1), f32),
            jax.ShapeDtypeStruct((B, 1), f32),
        ],
    )(wv, uv, pid2, lid2, nf8, dt2,
      w1w, w1u, pos_t, lang_t, w1p, w1l, w1n, b1r, w2r, b2r)

    return p2.reshape(B), h2.reshape(B)
